# Initial kernel scaffold; baseline (speedup 1.0000x reference)
import jax
import jax.numpy as jnp
from jax.experimental import pallas as pl


def kernel(x, input_pts, c1_centers, c1_W1, c1_W2, c1_Wl, c1_b, c3_centers, c3_W1, c3_W2, c3_Wl, c3_b, c4_centers, c4_W1, c4_W2, c4_Wl, c4_b, fcout_W, fcout2_W):
    raise NotImplementedError("write your pallas kernel here")


# speedup vs baseline: 1.0004x; 1.0004x over previous
import jax
import jax.numpy as jnp
from jax.experimental import pallas as pl
from jax.experimental.pallas import tpu as pltpu

N_CENTERS = 16
HIDDEN = 64


def _knn(out_pts, pts, K):
    d2 = (jnp.sum(out_pts ** 2, -1)[:, :, None]
          - 2.0 * jnp.einsum('bmd,bnd->bmn', out_pts, pts)
          + jnp.sum(pts ** 2, -1)[:, None, :])
    _, idx = jax.lax.top_k(-d2, K)
    return idx


def _composite_conv(x, pts, K, npts, centers, W1, W2, Wl, b):
    B = x.shape[0]
    out_pts = pts[:, :npts, :]
    idx = _knn(out_pts, pts, K)
    bidx = jnp.arange(B)[:, None, None]
    x_nb = x[bidx, idx]
    pts_nb = pts[bidx, idx]
    rel = pts_nb - out_pts[:, :, None, :]
    dmat = jnp.sqrt(jnp.sum((rel[:, :, :, None, :] - centers[None, None, None, :, :]) ** 2, -1) + 1e-12)
    w = jax.nn.relu(jnp.einsum('bmkc,ch->bmkh', dmat, W1))
    w = jnp.einsum('bmkh,hc->bmkc', w, W2)
    feat = jnp.einsum('bmkf,bmkc->bmcf', x_nb, w) / K
    out = feat.reshape(B, npts, -1) @ Wl + b
    return out, out_pts


def _apply_bn(v):
    B, M, C = v.shape
    vf = v.reshape(-1, C)
    mu = vf.mean(0)
    var = vf.var(0)
    return ((vf - mu) / jnp.sqrt(var + 1e-4)).reshape(B, M, C)


def _leaky(v):
    return jax.nn.leaky_relu(v, 0.1)


def _fc_head_kernel(x4_ref, fcW_ref, fc2W_ref, xout_ref, xreg_ref):
    x = x4_ref[...]
    xout = jnp.dot(x, fcW_ref[...], preferred_element_type=jnp.float32)
    xout_ref[...] = xout
    lk = jnp.where(xout > 0, xout, 0.1 * xout)
    xreg_ref[...] = jnp.dot(lk, fc2W_ref[...], preferred_element_type=jnp.float32)


def _fc_head(x4flat, fcout_W, fcout2_W):
    B = x4flat.shape[0]
    return pl.pallas_call(
        _fc_head_kernel,
        out_shape=(
            jax.ShapeDtypeStruct((B, fcout_W.shape[1]), jnp.float32),
            jax.ShapeDtypeStruct((B, fcout2_W.shape[1]), jnp.float32),
        ),
    )(x4flat, fcout_W, fcout2_W)


@jax.jit
def kernel(x, input_pts, c1_centers, c1_W1, c1_W2, c1_Wl, c1_b, c3_centers, c3_W1, c3_W2, c3_Wl, c3_b,
           c4_centers, c4_W1, c4_W2, c4_Wl, c4_b, fcout_W, fcout2_W):
    x1, p1 = _composite_conv(x, input_pts, 32, 128, c1_centers, c1_W1, c1_W2, c1_Wl, c1_b)
    x1 = _leaky(x1)
    x3, p3 = _composite_conv(x1, p1, 32, 32, c3_centers, c3_W1, c3_W2, c3_Wl, c3_b)
    x3 = _leaky(_apply_bn(x3))
    x4, p4 = _composite_conv(x3, p3, 32, 1, c4_centers, c4_W1, c4_W2, c4_Wl, c4_b)
    x4 = _leaky(x4)
    xout_flat = x4.reshape(x4.shape[0], -1)
    return _fc_head(xout_flat, fcout_W, fcout2_W)


# speedup vs baseline: 2.7734x; 2.7725x over previous
import functools

import jax
import jax.numpy as jnp
from jax import lax
from jax.experimental import pallas as pl
from jax.experimental.pallas import tpu as pltpu
from jax.experimental.pallas import tpu_sc as plsc

N_CENTERS = 16
HIDDEN = 64

_B = 32
_N1 = 16384
_M1 = 128
_M2 = 32
_K = 32
_BIG = jnp.float32(3.0e38)


def _bimerge(Ad, Ai, Bd, Bi):
    rBd = jnp.flip(Bd, 0)
    rBi = jnp.flip(Bi, 0)
    m = Ad <= rBd
    lod = jnp.where(m, Ad, rBd)
    loi = jnp.where(m, Ai, rBi)
    hid = jnp.where(m, rBd, Ad)
    hii = jnp.where(m, rBi, Ai)
    lod, loi = plsc.sort_key_val(lod, loi)
    hid, hii = plsc.sort_key_val(hid, hii)
    return lod, loi, hid, hii


def _knn_sc_body(px_h, py_h, pz_h, sp_h, qx_h, qy_h, qz_h, sq_h, o1_h, o2_h,
                 px_v, py_v, pz_v, sp_v, qx_v, qy_v, qz_v, sq_v, o1_v, o2_v):
    c = lax.axis_index("c")
    s = lax.axis_index("s")
    b = s * 2 + c
    pltpu.sync_copy(px_h.at[b], px_v)
    pltpu.sync_copy(py_h.at[b], py_v)
    pltpu.sync_copy(pz_h.at[b], pz_v)
    pltpu.sync_copy(sp_h.at[b], sp_v)
    pltpu.sync_copy(qx_h.at[b], qx_v)
    pltpu.sync_copy(qy_h.at[b], qy_v)
    pltpu.sync_copy(qz_h.at[b], qz_v)
    pltpu.sync_copy(sq_h.at[b], sq_v)

    def chunk_d2(j, qxv, qyv, qzv, sqv):
        off = pl.multiple_of(j * 16, 16)
        m = qxv * px_v[pl.ds(off, 16)]
        m = m + qyv * py_v[pl.ds(off, 16)]
        m = m + qzv * pz_v[pl.ds(off, 16)]
        d2 = (sqv - (m + m)) + sp_v[pl.ds(off, 16)]
        ci = lax.iota(jnp.int32, 16) + j * 16
        return d2, ci

    def run_layer(n_chunks, n_queries, out_v):
        def per_query(mq, _):
            qoff = pl.multiple_of(mq * 16, 16)
            qxv = qx_v[pl.ds(qoff, 16)]
            qyv = qy_v[pl.ds(qoff, 16)]
            qzv = qz_v[pl.ds(qoff, 16)]
            sqv = sq_v[pl.ds(qoff, 16)]
            d0, i0 = chunk_d2(0, qxv, qyv, qzv, sqv)
            d0, i0 = plsc.sort_key_val(d0, i0)
            d1, i1 = chunk_d2(1, qxv, qyv, qzv, sqv)
            d1, i1 = plsc.sort_key_val(d1, i1)
            B0d, B0i, B1d, B1i = _bimerge(d0, i0, d1, i1)
            tau = jnp.max(B1d)

            def scan_chunk(j, carry):
                B0d, B0i, B1d, B1i, tau = carry
                d2, ci = chunk_d2(j, qxv, qyv, qzv, sqv)
                dmin = jnp.min(d2)

                def do_merge(args):
                    B0d, B0i, B1d, B1i = args
                    tauv = jnp.full((16,), tau, jnp.float32)
                    dm = jnp.where(d2 < tauv, d2, _BIG)
                    Cd, Ci = plsc.sort_key_val(dm, ci)
                    nB0d, nB0i, h0d, h0i = _bimerge(B0d, B0i, Cd, Ci)
                    nB1d, nB1i, _, _ = _bimerge(B1d, B1i, h0d, h0i)
                    return nB0d, nB0i, nB1d, nB1i, jnp.max(nB1d)

                def no_merge(args):
                    B0d, B0i, B1d, B1i = args
                    return B0d, B0i, B1d, B1i, tau

                return lax.cond(dmin < tau, do_merge, no_merge,
                                (B0d, B0i, B1d, B1i))

            B0d, B0i, B1d, B1i, tau = lax.fori_loop(
                2, n_chunks, scan_chunk, (B0d, B0i, B1d, B1i, tau))
            ooff = pl.multiple_of(mq * 32, 32)
            out_v[pl.ds(ooff, 16)] = B0i
            out_v[pl.ds(ooff + 16, 16)] = B1i
            return 0

        lax.fori_loop(0, n_queries, per_query, 0)

    run_layer(_N1 // 16, _M1, o1_v)
    run_layer(_M1 // 16, _M2, o2_v)
    pltpu.sync_copy(o1_v, o1_h.at[b])
    pltpu.sync_copy(o2_v, o2_h.at[b])


def _bf16_round(p):
    r = lax.bitcast_convert_type(p, jnp.uint32)
    r = (r + jnp.uint32(0x7FFF) + ((r >> 16) & jnp.uint32(1))) & jnp.uint32(0xFFFF0000)
    return lax.bitcast_convert_type(r, jnp.float32)


def _knn_sc(input_pts):
    ptsr = _bf16_round(input_pts)
    px = ptsr[:, :, 0]
    py = ptsr[:, :, 1]
    pz = ptsr[:, :, 2]
    sump = jnp.sum(input_pts ** 2, -1)
    q = ptsr[:, :_M1, :]
    qx = jnp.broadcast_to(q[:, :, 0:1], (_B, _M1, 16)).reshape(_B, _M1 * 16)
    qy = jnp.broadcast_to(q[:, :, 1:2], (_B, _M1, 16)).reshape(_B, _M1 * 16)
    qz = jnp.broadcast_to(q[:, :, 2:3], (_B, _M1, 16)).reshape(_B, _M1 * 16)
    sumq = jnp.sum(input_pts[:, :_M1, :] ** 2, -1)
    sq = jnp.broadcast_to(sumq[:, :, None], (_B, _M1, 16)).reshape(_B, _M1 * 16)

    mesh = plsc.VectorSubcoreMesh(core_axis_name="c", subcore_axis_name="s")
    o1, o2 = pl.kernel(
        _knn_sc_body,
        out_type=(
            jax.ShapeDtypeStruct((_B, _M1 * _K), jnp.int32),
            jax.ShapeDtypeStruct((_B, _M2 * _K), jnp.int32),
        ),
        mesh=mesh,
        compiler_params=pltpu.CompilerParams(needs_layout_passes=False),
        scratch_types=[
            pltpu.VMEM((_N1,), jnp.float32),
            pltpu.VMEM((_N1,), jnp.float32),
            pltpu.VMEM((_N1,), jnp.float32),
            pltpu.VMEM((_N1,), jnp.float32),
            pltpu.VMEM((_M1 * 16,), jnp.float32),
            pltpu.VMEM((_M1 * 16,), jnp.float32),
            pltpu.VMEM((_M1 * 16,), jnp.float32),
            pltpu.VMEM((_M1 * 16,), jnp.float32),
            pltpu.VMEM((_M1 * _K,), jnp.int32),
            pltpu.VMEM((_M2 * _K,), jnp.int32),
        ],
    )(px, py, pz, sump, qx, qy, qz, sq)
    return o1.reshape(_B, _M1, _K), o2.reshape(_B, _M2, _K)


def _conv_dense(x_nb, pts_nb, out_pts, centers, W1, W2, Wl, b, K):
    B, M = x_nb.shape[0], x_nb.shape[1]
    rel = pts_nb - out_pts[:, :, None, :]
    dmat = jnp.sqrt(jnp.sum((rel[:, :, :, None, :] - centers[None, None, None, :, :]) ** 2, -1) + 1e-12)
    w = jax.nn.relu(jnp.einsum('bmkc,ch->bmkh', dmat, W1))
    w = jnp.einsum('bmkh,hc->bmkc', w, W2)
    feat = jnp.einsum('bmkf,bmkc->bmcf', x_nb, w) / K
    return feat.reshape(B, M, -1) @ Wl + b


def _apply_bn(v):
    B, M, C = v.shape
    vf = v.reshape(-1, C)
    mu = vf.mean(0)
    var = vf.var(0)
    return ((vf - mu) / jnp.sqrt(var + 1e-4)).reshape(B, M, C)


def _leaky(v):
    return jax.nn.leaky_relu(v, 0.1)


def _fc_head_kernel(x4_ref, fcW_ref, fc2W_ref, xout_ref, xreg_ref):
    x = x4_ref[...]
    xout = jnp.dot(x, fcW_ref[...], preferred_element_type=jnp.float32)
    xout_ref[...] = xout
    lk = jnp.where(xout > 0, xout, 0.1 * xout)
    xreg_ref[...] = jnp.dot(lk, fc2W_ref[...], preferred_element_type=jnp.float32)


def _fc_head(x4flat, fcout_W, fcout2_W):
    B = x4flat.shape[0]
    return pl.pallas_call(
        _fc_head_kernel,
        out_shape=(
            jax.ShapeDtypeStruct((B, fcout_W.shape[1]), jnp.float32),
            jax.ShapeDtypeStruct((B, fcout2_W.shape[1]), jnp.float32),
        ),
    )(x4flat, fcout_W, fcout2_W)


@jax.jit
def kernel(x, input_pts, c1_centers, c1_W1, c1_W2, c1_Wl, c1_b, c3_centers, c3_W1, c3_W2, c3_Wl, c3_b,
           c4_centers, c4_W1, c4_W2, c4_Wl, c4_b, fcout_W, fcout2_W):
    idx1, idx2 = _knn_sc(input_pts)
    bidx = jnp.arange(_B)[:, None, None]

    p1 = input_pts[:, :_M1, :]
    x_nb = x[bidx, idx1]
    pts_nb = input_pts[bidx, idx1]
    x1 = _leaky(_conv_dense(x_nb, pts_nb, p1, c1_centers, c1_W1, c1_W2, c1_Wl, c1_b, _K))

    p3 = p1[:, :_M2, :]
    x_nb2 = x1[bidx, idx2]
    pts_nb2 = p1[bidx, idx2]
    x3 = _conv_dense(x_nb2, pts_nb2, p3, c3_centers, c3_W1, c3_W2, c3_Wl, c3_b, _K)
    x3 = _leaky(_apply_bn(x3))

    p4 = p3[:, :1, :]
    x_nb3 = x3[:, None, :, :]
    pts_nb3 = p3[:, None, :, :]
    x4 = _leaky(_conv_dense(x_nb3, pts_nb3, p4, c4_centers, c4_W1, c4_W2, c4_Wl, c4_b, _K))

    xout_flat = x4.reshape(x4.shape[0], -1)
    return _fc_head(xout_flat, fcout_W, fcout2_W)


# speedup vs baseline: 3.6204x; 1.3054x over previous
import functools

import jax
import jax.numpy as jnp
from jax import lax
from jax.experimental import pallas as pl
from jax.experimental.pallas import tpu as pltpu
from jax.experimental.pallas import tpu_sc as plsc

N_CENTERS = 16
HIDDEN = 64

_B = 32
_N1 = 16384
_M1 = 128
_M2 = 32
_K = 32
_BIG = jnp.float32(3.0e38)


def _bimerge(Ad, Ai, Bd, Bi):
    rBd = jnp.flip(Bd, 0)
    rBi = jnp.flip(Bi, 0)
    m = Ad <= rBd
    lod = jnp.where(m, Ad, rBd)
    loi = jnp.where(m, Ai, rBi)
    hid = jnp.where(m, rBd, Ad)
    hii = jnp.where(m, rBi, Ai)
    lod, loi = plsc.sort_key_val(lod, loi)
    hid, hii = plsc.sort_key_val(hid, hii)
    return lod, loi, hid, hii


def _knn_sc_body(px_h, py_h, pz_h, sp_h, qx_h, qy_h, qz_h, sq_h, o1_h, o2_h,
                 px_v, py_v, pz_v, sp_v, qx_v, qy_v, qz_v, sq_v, o1_v, o2_v):
    c = lax.axis_index("c")
    s = lax.axis_index("s")
    b = s * 2 + c
    pltpu.sync_copy(px_h.at[b], px_v)
    pltpu.sync_copy(py_h.at[b], py_v)
    pltpu.sync_copy(pz_h.at[b], pz_v)
    pltpu.sync_copy(sp_h.at[b], sp_v)
    pltpu.sync_copy(qx_h.at[b], qx_v)
    pltpu.sync_copy(qy_h.at[b], qy_v)
    pltpu.sync_copy(qz_h.at[b], qz_v)
    pltpu.sync_copy(sq_h.at[b], sq_v)

    def chunk_d2(j, qxv, qyv, qzv, sqv):
        off = pl.multiple_of(j * 16, 16)
        m = qxv * px_v[pl.ds(off, 16)]
        m = m + qyv * py_v[pl.ds(off, 16)]
        m = m + qzv * pz_v[pl.ds(off, 16)]
        d2 = (sqv - (m + m)) + sp_v[pl.ds(off, 16)]
        ci = lax.iota(jnp.int32, 16) + j * 16
        return d2, ci

    def run_layer(n_chunks, n_queries, out_v):
        def per_query(mq, _):
            qoff = pl.multiple_of(mq * 16, 16)
            qxv = qx_v[pl.ds(qoff, 16)]
            qyv = qy_v[pl.ds(qoff, 16)]
            qzv = qz_v[pl.ds(qoff, 16)]
            sqv = sq_v[pl.ds(qoff, 16)]
            d0, i0 = chunk_d2(0, qxv, qyv, qzv, sqv)
            d0, i0 = plsc.sort_key_val(d0, i0)
            d1, i1 = chunk_d2(1, qxv, qyv, qzv, sqv)
            d1, i1 = plsc.sort_key_val(d1, i1)
            B0d, B0i, B1d, B1i = _bimerge(d0, i0, d1, i1)
            tau = jnp.max(B1d)

            def scan_chunk(j, carry):
                B0d, B0i, B1d, B1i, tau = carry
                d2, ci = chunk_d2(j, qxv, qyv, qzv, sqv)
                dmin = jnp.min(d2)

                def do_merge(args):
                    B0d, B0i, B1d, B1i = args
                    tauv = jnp.full((16,), tau, jnp.float32)
                    dm = jnp.where(d2 < tauv, d2, _BIG)
                    Cd, Ci = plsc.sort_key_val(dm, ci)
                    nB0d, nB0i, h0d, h0i = _bimerge(B0d, B0i, Cd, Ci)
                    nB1d, nB1i, _, _ = _bimerge(B1d, B1i, h0d, h0i)
                    return nB0d, nB0i, nB1d, nB1i, jnp.max(nB1d)

                def no_merge(args):
                    B0d, B0i, B1d, B1i = args
                    return B0d, B0i, B1d, B1i, tau

                return lax.cond(dmin < tau, do_merge, no_merge,
                                (B0d, B0i, B1d, B1i))

            carry = lax.fori_loop(
                2, min(8, n_chunks), scan_chunk, (B0d, B0i, B1d, B1i, tau))

            def scan_group(g, carry):
                tau = carry[4]
                base = g * 8
                acc, _ = chunk_d2(base, qxv, qyv, qzv, sqv)
                for t in range(1, 8):
                    dt, _ = chunk_d2(base + t, qxv, qyv, qzv, sqv)
                    acc = jnp.minimum(acc, dt)
                gmin = jnp.min(acc)

                def hit(c):
                    return lax.fori_loop(base, base + 8, scan_chunk, c)

                return lax.cond(gmin < tau, hit, lambda c: c, carry)

            if n_chunks > 8:
                carry = lax.fori_loop(1, n_chunks // 8, scan_group, carry)
            B0d, B0i, B1d, B1i, tau = carry
            ooff = pl.multiple_of(mq * 32, 32)
            out_v[pl.ds(ooff, 16)] = B0i
            out_v[pl.ds(ooff + 16, 16)] = B1i
            return 0

        lax.fori_loop(0, n_queries, per_query, 0)

    run_layer(_N1 // 16, _M1, o1_v)
    run_layer(_M1 // 16, _M2, o2_v)
    pltpu.sync_copy(o1_v, o1_h.at[b])
    pltpu.sync_copy(o2_v, o2_h.at[b])


def _bf16_round(p):
    r = lax.bitcast_convert_type(p, jnp.uint32)
    r = (r + jnp.uint32(0x7FFF) + ((r >> 16) & jnp.uint32(1))) & jnp.uint32(0xFFFF0000)
    return lax.bitcast_convert_type(r, jnp.float32)


def _knn_sc(input_pts):
    ptsr = _bf16_round(input_pts)
    px = ptsr[:, :, 0]
    py = ptsr[:, :, 1]
    pz = ptsr[:, :, 2]
    sump = jnp.sum(input_pts ** 2, -1)
    q = ptsr[:, :_M1, :]
    qx = jnp.broadcast_to(q[:, :, 0:1], (_B, _M1, 16)).reshape(_B, _M1 * 16)
    qy = jnp.broadcast_to(q[:, :, 1:2], (_B, _M1, 16)).reshape(_B, _M1 * 16)
    qz = jnp.broadcast_to(q[:, :, 2:3], (_B, _M1, 16)).reshape(_B, _M1 * 16)
    sumq = jnp.sum(input_pts[:, :_M1, :] ** 2, -1)
    sq = jnp.broadcast_to(sumq[:, :, None], (_B, _M1, 16)).reshape(_B, _M1 * 16)

    mesh = plsc.VectorSubcoreMesh(core_axis_name="c", subcore_axis_name="s")
    o1, o2 = pl.kernel(
        _knn_sc_body,
        out_type=(
            jax.ShapeDtypeStruct((_B, _M1 * _K), jnp.int32),
            jax.ShapeDtypeStruct((_B, _M2 * _K), jnp.int32),
        ),
        mesh=mesh,
        compiler_params=pltpu.CompilerParams(needs_layout_passes=False),
        scratch_types=[
            pltpu.VMEM((_N1,), jnp.float32),
            pltpu.VMEM((_N1,), jnp.float32),
            pltpu.VMEM((_N1,), jnp.float32),
            pltpu.VMEM((_N1,), jnp.float32),
            pltpu.VMEM((_M1 * 16,), jnp.float32),
            pltpu.VMEM((_M1 * 16,), jnp.float32),
            pltpu.VMEM((_M1 * 16,), jnp.float32),
            pltpu.VMEM((_M1 * 16,), jnp.float32),
            pltpu.VMEM((_M1 * _K,), jnp.int32),
            pltpu.VMEM((_M2 * _K,), jnp.int32),
        ],
    )(px, py, pz, sump, qx, qy, qz, sq)
    return o1.reshape(_B, _M1, _K), o2.reshape(_B, _M2, _K)


def _conv_dense(x_nb, pts_nb, out_pts, centers, W1, W2, Wl, b, K):
    B, M = x_nb.shape[0], x_nb.shape[1]
    rel = pts_nb - out_pts[:, :, None, :]
    dmat = jnp.sqrt(jnp.sum((rel[:, :, :, None, :] - centers[None, None, None, :, :]) ** 2, -1) + 1e-12)
    w = jax.nn.relu(jnp.einsum('bmkc,ch->bmkh', dmat, W1))
    w = jnp.einsum('bmkh,hc->bmkc', w, W2)
    feat = jnp.einsum('bmkf,bmkc->bmcf', x_nb, w) / K
    return feat.reshape(B, M, -1) @ Wl + b


def _apply_bn(v):
    B, M, C = v.shape
    vf = v.reshape(-1, C)
    mu = vf.mean(0)
    var = vf.var(0)
    return ((vf - mu) / jnp.sqrt(var + 1e-4)).reshape(B, M, C)


def _leaky(v):
    return jax.nn.leaky_relu(v, 0.1)


def _fc_head_kernel(x4_ref, fcW_ref, fc2W_ref, xout_ref, xreg_ref):
    x = x4_ref[...]
    xout = jnp.dot(x, fcW_ref[...], preferred_element_type=jnp.float32)
    xout_ref[...] = xout
    lk = jnp.where(xout > 0, xout, 0.1 * xout)
    xreg_ref[...] = jnp.dot(lk, fc2W_ref[...], preferred_element_type=jnp.float32)


def _fc_head(x4flat, fcout_W, fcout2_W):
    B = x4flat.shape[0]
    return pl.pallas_call(
        _fc_head_kernel,
        out_shape=(
            jax.ShapeDtypeStruct((B, fcout_W.shape[1]), jnp.float32),
            jax.ShapeDtypeStruct((B, fcout2_W.shape[1]), jnp.float32),
        ),
    )(x4flat, fcout_W, fcout2_W)


@jax.jit
def kernel(x, input_pts, c1_centers, c1_W1, c1_W2, c1_Wl, c1_b, c3_centers, c3_W1, c3_W2, c3_Wl, c3_b,
           c4_centers, c4_W1, c4_W2, c4_Wl, c4_b, fcout_W, fcout2_W):
    idx1, idx2 = _knn_sc(input_pts)
    bidx = jnp.arange(_B)[:, None, None]

    p1 = input_pts[:, :_M1, :]
    x_nb = x[bidx, idx1]
    pts_nb = input_pts[bidx, idx1]
    x1 = _leaky(_conv_dense(x_nb, pts_nb, p1, c1_centers, c1_W1, c1_W2, c1_Wl, c1_b, _K))

    p3 = p1[:, :_M2, :]
    x_nb2 = x1[bidx, idx2]
    pts_nb2 = p1[bidx, idx2]
    x3 = _conv_dense(x_nb2, pts_nb2, p3, c3_centers, c3_W1, c3_W2, c3_Wl, c3_b, _K)
    x3 = _leaky(_apply_bn(x3))

    p4 = p3[:, :1, :]
    x_nb3 = x3[:, None, :, :]
    pts_nb3 = p3[:, None, :, :]
    x4 = _leaky(_conv_dense(x_nb3, pts_nb3, p4, c4_centers, c4_W1, c4_W2, c4_Wl, c4_b, _K))

    xout_flat = x4.reshape(x4.shape[0], -1)
    return _fc_head(xout_flat, fcout_W, fcout2_W)


# speedup vs baseline: 5.2998x; 1.4639x over previous
import functools

import jax
import jax.numpy as jnp
from jax import lax
from jax.experimental import pallas as pl
from jax.experimental.pallas import tpu as pltpu
from jax.experimental.pallas import tpu_sc as plsc

N_CENTERS = 16
HIDDEN = 64

_B = 32
_N1 = 16384
_M1 = 128
_M2 = 32
_K = 32
_BIG = jnp.float32(3.0e38)


def _bimerge(Ad, Ai, Bd, Bi):
    rBd = jnp.flip(Bd, 0)
    rBi = jnp.flip(Bi, 0)
    m = Ad <= rBd
    lod = jnp.where(m, Ad, rBd)
    loi = jnp.where(m, Ai, rBi)
    hid = jnp.where(m, rBd, Ad)
    hii = jnp.where(m, rBi, Ai)
    lod, loi = plsc.sort_key_val(lod, loi)
    hid, hii = plsc.sort_key_val(hid, hii)
    return lod, loi, hid, hii


def _knn_sc_body(px_h, py_h, pz_h, sp_h, qx_h, qy_h, qz_h, sq_h, o1_h, o2_h,
                 px_v, py_v, pz_v, sp_v, qx_v, qy_v, qz_v, sq_v, o1_v, o2_v):
    c = lax.axis_index("c")
    s = lax.axis_index("s")
    b = s * 2 + c
    pltpu.sync_copy(px_h.at[b], px_v)
    pltpu.sync_copy(py_h.at[b], py_v)
    pltpu.sync_copy(pz_h.at[b], pz_v)
    pltpu.sync_copy(sp_h.at[b], sp_v)
    pltpu.sync_copy(qx_h.at[b], qx_v)
    pltpu.sync_copy(qy_h.at[b], qy_v)
    pltpu.sync_copy(qz_h.at[b], qz_v)
    pltpu.sync_copy(sq_h.at[b], sq_v)

    def chunk_d2(j, qxv, qyv, qzv, sqv):
        off = pl.multiple_of(j * 16, 16)
        m = qxv * px_v[pl.ds(off, 16)]
        m = m + qyv * py_v[pl.ds(off, 16)]
        m = m + qzv * pz_v[pl.ds(off, 16)]
        d2 = (sqv - (m + m)) + sp_v[pl.ds(off, 16)]
        ci = lax.iota(jnp.int32, 16) + j * 16
        return d2, ci

    def run_layer(n_chunks, n_queries, out_v):
        def per_query(mq, _):
            qoff = pl.multiple_of(mq * 16, 16)
            qxv = qx_v[pl.ds(qoff, 16)]
            qyv = qy_v[pl.ds(qoff, 16)]
            qzv = qz_v[pl.ds(qoff, 16)]
            sqv = sq_v[pl.ds(qoff, 16)]
            d0, i0 = chunk_d2(0, qxv, qyv, qzv, sqv)
            d0, i0 = plsc.sort_key_val(d0, i0)
            d1, i1 = chunk_d2(1, qxv, qyv, qzv, sqv)
            d1, i1 = plsc.sort_key_val(d1, i1)
            B0d, B0i, B1d, B1i = _bimerge(d0, i0, d1, i1)
            tau = jnp.max(B1d)

            def scan_chunk(j, carry):
                B0d, B0i, B1d, B1i, tau = carry
                d2, ci = chunk_d2(j, qxv, qyv, qzv, sqv)
                dmin = jnp.min(d2)

                def do_merge(args):
                    B0d, B0i, B1d, B1i = args
                    tauv = jnp.full((16,), tau, jnp.float32)
                    dm = jnp.where(d2 < tauv, d2, _BIG)
                    Cd, Ci = plsc.sort_key_val(dm, ci)
                    nB0d, nB0i, h0d, h0i = _bimerge(B0d, B0i, Cd, Ci)
                    nB1d, nB1i, _, _ = _bimerge(B1d, B1i, h0d, h0i)
                    return nB0d, nB0i, nB1d, nB1i, jnp.max(nB1d)

                def no_merge(args):
                    B0d, B0i, B1d, B1i = args
                    return B0d, B0i, B1d, B1i, tau

                return lax.cond(dmin < tau, do_merge, no_merge,
                                (B0d, B0i, B1d, B1i))

            carry = lax.fori_loop(
                2, min(8, n_chunks), scan_chunk, (B0d, B0i, B1d, B1i, tau))

            def scan_group(g, carry):
                tau = carry[4]
                tauv = jnp.full((16,), tau, jnp.float32)
                base = g * 8
                acc, acci = chunk_d2(base, qxv, qyv, qzv, sqv)
                cnt = (acc < tauv).astype(jnp.int32)
                for t in range(1, 8):
                    dt, it = chunk_d2(base + t, qxv, qyv, qzv, sqv)
                    m = dt < acc
                    acc = jnp.where(m, dt, acc)
                    acci = jnp.where(m, it, acci)
                    cnt = cnt + (dt < tauv).astype(jnp.int32)
                gmin = jnp.min(acc)

                def hit(c):
                    cmax = jnp.max(cnt)

                    def merge_lane_mins(c):
                        B0d, B0i, B1d, B1i, tau = c
                        dm = jnp.where(acc < tauv, acc, _BIG)
                        Cd, Ci = plsc.sort_key_val(dm, acci)
                        nB0d, nB0i, h0d, h0i = _bimerge(B0d, B0i, Cd, Ci)
                        nB1d, nB1i, _, _ = _bimerge(B1d, B1i, h0d, h0i)
                        return nB0d, nB0i, nB1d, nB1i, jnp.max(nB1d)

                    def rescan(c):
                        return lax.fori_loop(base, base + 8, scan_chunk, c)

                    return lax.cond(cmax <= 1, merge_lane_mins, rescan, c)

                return lax.cond(gmin < tau, hit, lambda c: c, carry)

            if n_chunks > 8:
                carry = lax.fori_loop(1, n_chunks // 8, scan_group, carry)
            B0d, B0i, B1d, B1i, tau = carry
            ooff = pl.multiple_of(mq * 32, 32)
            out_v[pl.ds(ooff, 16)] = B0i
            out_v[pl.ds(ooff + 16, 16)] = B1i
            return 0

        lax.fori_loop(0, n_queries, per_query, 0)

    run_layer(_N1 // 16, _M1, o1_v)
    run_layer(_M1 // 16, _M2, o2_v)
    pltpu.sync_copy(o1_v, o1_h.at[b])
    pltpu.sync_copy(o2_v, o2_h.at[b])


def _bf16_round(p):
    r = lax.bitcast_convert_type(p, jnp.uint32)
    r = (r + jnp.uint32(0x7FFF) + ((r >> 16) & jnp.uint32(1))) & jnp.uint32(0xFFFF0000)
    return lax.bitcast_convert_type(r, jnp.float32)


def _knn_sc(input_pts):
    ptsr = _bf16_round(input_pts)
    px = ptsr[:, :, 0]
    py = ptsr[:, :, 1]
    pz = ptsr[:, :, 2]
    sump = jnp.sum(input_pts ** 2, -1)
    q = ptsr[:, :_M1, :]
    qx = jnp.broadcast_to(q[:, :, 0:1], (_B, _M1, 16)).reshape(_B, _M1 * 16)
    qy = jnp.broadcast_to(q[:, :, 1:2], (_B, _M1, 16)).reshape(_B, _M1 * 16)
    qz = jnp.broadcast_to(q[:, :, 2:3], (_B, _M1, 16)).reshape(_B, _M1 * 16)
    sumq = jnp.sum(input_pts[:, :_M1, :] ** 2, -1)
    sq = jnp.broadcast_to(sumq[:, :, None], (_B, _M1, 16)).reshape(_B, _M1 * 16)

    mesh = plsc.VectorSubcoreMesh(core_axis_name="c", subcore_axis_name="s")
    o1, o2 = pl.kernel(
        _knn_sc_body,
        out_type=(
            jax.ShapeDtypeStruct((_B, _M1 * _K), jnp.int32),
            jax.ShapeDtypeStruct((_B, _M2 * _K), jnp.int32),
        ),
        mesh=mesh,
        compiler_params=pltpu.CompilerParams(needs_layout_passes=False),
        scratch_types=[
            pltpu.VMEM((_N1,), jnp.float32),
            pltpu.VMEM((_N1,), jnp.float32),
            pltpu.VMEM((_N1,), jnp.float32),
            pltpu.VMEM((_N1,), jnp.float32),
            pltpu.VMEM((_M1 * 16,), jnp.float32),
            pltpu.VMEM((_M1 * 16,), jnp.float32),
            pltpu.VMEM((_M1 * 16,), jnp.float32),
            pltpu.VMEM((_M1 * 16,), jnp.float32),
            pltpu.VMEM((_M1 * _K,), jnp.int32),
            pltpu.VMEM((_M2 * _K,), jnp.int32),
        ],
    )(px, py, pz, sump, qx, qy, qz, sq)
    return o1.reshape(_B, _M1, _K), o2.reshape(_B, _M2, _K)


def _conv_dense(x_nb, pts_nb, out_pts, centers, W1, W2, Wl, b, K):
    B, M = x_nb.shape[0], x_nb.shape[1]
    rel = pts_nb - out_pts[:, :, None, :]
    dmat = jnp.sqrt(jnp.sum((rel[:, :, :, None, :] - centers[None, None, None, :, :]) ** 2, -1) + 1e-12)
    w = jax.nn.relu(jnp.einsum('bmkc,ch->bmkh', dmat, W1))
    w = jnp.einsum('bmkh,hc->bmkc', w, W2)
    feat = jnp.einsum('bmkf,bmkc->bmcf', x_nb, w) / K
    return feat.reshape(B, M, -1) @ Wl + b


def _apply_bn(v):
    B, M, C = v.shape
    vf = v.reshape(-1, C)
    mu = vf.mean(0)
    var = vf.var(0)
    return ((vf - mu) / jnp.sqrt(var + 1e-4)).reshape(B, M, C)


def _leaky(v):
    return jax.nn.leaky_relu(v, 0.1)


def _fc_head_kernel(x4_ref, fcW_ref, fc2W_ref, xout_ref, xreg_ref):
    x = x4_ref[...]
    xout = jnp.dot(x, fcW_ref[...], preferred_element_type=jnp.float32)
    xout_ref[...] = xout
    lk = jnp.where(xout > 0, xout, 0.1 * xout)
    xreg_ref[...] = jnp.dot(lk, fc2W_ref[...], preferred_element_type=jnp.float32)


def _fc_head(x4flat, fcout_W, fcout2_W):
    B = x4flat.shape[0]
    return pl.pallas_call(
        _fc_head_kernel,
        out_shape=(
            jax.ShapeDtypeStruct((B, fcout_W.shape[1]), jnp.float32),
            jax.ShapeDtypeStruct((B, fcout2_W.shape[1]), jnp.float32),
        ),
    )(x4flat, fcout_W, fcout2_W)


@jax.jit
def kernel(x, input_pts, c1_centers, c1_W1, c1_W2, c1_Wl, c1_b, c3_centers, c3_W1, c3_W2, c3_Wl, c3_b,
           c4_centers, c4_W1, c4_W2, c4_Wl, c4_b, fcout_W, fcout2_W):
    idx1, idx2 = _knn_sc(input_pts)
    bidx = jnp.arange(_B)[:, None, None]

    p1 = input_pts[:, :_M1, :]
    x_nb = x[bidx, idx1]
    pts_nb = input_pts[bidx, idx1]
    x1 = _leaky(_conv_dense(x_nb, pts_nb, p1, c1_centers, c1_W1, c1_W2, c1_Wl, c1_b, _K))

    p3 = p1[:, :_M2, :]
    x_nb2 = x1[bidx, idx2]
    pts_nb2 = p1[bidx, idx2]
    x3 = _conv_dense(x_nb2, pts_nb2, p3, c3_centers, c3_W1, c3_W2, c3_Wl, c3_b, _K)
    x3 = _leaky(_apply_bn(x3))

    p4 = p3[:, :1, :]
    x_nb3 = x3[:, None, :, :]
    pts_nb3 = p3[:, None, :, :]
    x4 = _leaky(_conv_dense(x_nb3, pts_nb3, p4, c4_centers, c4_W1, c4_W2, c4_Wl, c4_b, _K))

    xout_flat = x4.reshape(x4.shape[0], -1)
    return _fc_head(xout_flat, fcout_W, fcout2_W)
